# baseline (speedup 1.0000x reference)
import jax
import jax.numpy as jnp
from jax.experimental import pallas as pl
from jax.experimental.pallas import tpu as pltpu

_N = 10000
_F = 128
_BM = 512


def _fc1_kernel(x_ref, w1_ref, b1_ref, y_ref):
    y_ref[...] = jax.lax.dot_general(
        x_ref[...], w1_ref[...], (((1,), (1,)), ((), ())),
        preferred_element_type=jnp.float32) + b1_ref[...]


def _layer1_kernel(a_ref, y1_ref, w2_ref, b2_ref, y2_ref):
    h = jnp.dot(a_ref[...].astype(jnp.bfloat16),
                y1_ref[...].astype(jnp.bfloat16),
                preferred_element_type=jnp.float32)
    h = jnp.maximum(h, 0.0)
    y2_ref[...] = jax.lax.dot_general(
        h, w2_ref[...], (((1,), (1,)), ((), ())),
        preferred_element_type=jnp.float32) + b2_ref[...]


def _layer2_kernel(a_ref, y2_ref, out_ref):
    out_ref[...] = jnp.dot(a_ref[...].astype(jnp.bfloat16),
                           y2_ref[...].astype(jnp.bfloat16),
                           preferred_element_type=jnp.float32)


def kernel(X, A, W1, b1, W2, b2):
    b1r = b1.reshape(1, _F)
    b2r = b2.reshape(1, _F)

    y1 = pl.pallas_call(
        _fc1_kernel,
        out_shape=jax.ShapeDtypeStruct((_N, _F), jnp.float32),
    )(X, W1, b1r)

    grid = (pl.cdiv(_N, _BM),)
    y2 = pl.pallas_call(
        _layer1_kernel,
        grid=grid,
        in_specs=[
            pl.BlockSpec((_BM, _N), lambda i: (i, 0)),
            pl.BlockSpec((_N, _F), lambda i: (0, 0)),
            pl.BlockSpec((_F, _F), lambda i: (0, 0)),
            pl.BlockSpec((1, _F), lambda i: (0, 0)),
        ],
        out_specs=pl.BlockSpec((_BM, _F), lambda i: (i, 0)),
        out_shape=jax.ShapeDtypeStruct((_N, _F), jnp.float32),
        compiler_params=pltpu.CompilerParams(
            vmem_limit_bytes=100 * 1024 * 1024),
    )(A, y1, W2, b2r)

    out = pl.pallas_call(
        _layer2_kernel,
        grid=grid,
        in_specs=[
            pl.BlockSpec((_BM, _N), lambda i: (i, 0)),
            pl.BlockSpec((_N, _F), lambda i: (0, 0)),
        ],
        out_specs=pl.BlockSpec((_BM, _F), lambda i: (i, 0)),
        out_shape=jax.ShapeDtypeStruct((_N, _F), jnp.float32),
        compiler_params=pltpu.CompilerParams(
            vmem_limit_bytes=100 * 1024 * 1024),
    )(A, y2)
    return out


# speedup vs baseline: 1.0495x; 1.0495x over previous
import jax
import jax.numpy as jnp
from jax.experimental import pallas as pl
from jax.experimental.pallas import tpu as pltpu

_N = 10000
_F = 128
_BM = 200
_NB = _N // _BM
_S = 9


def _fc1_kernel(x_ref, w1_ref, b1_ref, y_ref):
    y1 = jax.lax.dot_general(
        x_ref[...].astype(jnp.bfloat16), w1_ref[...].astype(jnp.bfloat16),
        (((1,), (1,)), ((), ())), preferred_element_type=jnp.float32)
    y_ref[...] = (y1 + b1_ref[...]).astype(jnp.bfloat16)


def _gcn_kernel(a_ref, y1_ref, w2_ref, b2_ref, out_ref, y2_ref, cache_ref):
    i = pl.program_id(0)

    @pl.when(i < _NB)
    def _pass1():
        a_bf = a_ref[...].astype(jnp.bfloat16)
        h = jnp.dot(a_bf, y1_ref[...], preferred_element_type=jnp.float32)
        h = jnp.maximum(h, 0.0).astype(jnp.bfloat16)
        w2b = w2_ref[...].astype(jnp.bfloat16)
        y2 = jax.lax.dot_general(h, w2b, (((1,), (1,)), ((), ())),
                                 preferred_element_type=jnp.float32)
        y2 = y2 + b2_ref[...]
        y2_ref[pl.ds(i * _BM, _BM), :] = y2.astype(jnp.bfloat16)

        slot = i - (_NB - 1 - _S)

        @pl.when((slot >= 0) & (i <= _NB - 2))
        def _store_cache():
            cache_ref[slot] = a_bf

    @pl.when(i >= _NB)
    def _pass2():
        t = i - _NB
        from_cache = (t >= 1) & (t <= _S)

        @pl.when(from_cache)
        def _cached():
            a_bf = cache_ref[_S - t]
            out_ref[...] = jnp.dot(a_bf, y2_ref[...],
                                   preferred_element_type=jnp.float32)

        @pl.when(jnp.logical_not(from_cache))
        def _streamed():
            a_bf = a_ref[...].astype(jnp.bfloat16)
            out_ref[...] = jnp.dot(a_bf, y2_ref[...],
                                   preferred_element_type=jnp.float32)


def _a_index(i):
    t = i - _NB
    j = _NB - 1 - t
    pinned = (t >= 0) & (t <= _S)
    pass2_idx = jnp.where(pinned, _NB - 1, j)
    return (jnp.where(i < _NB, i, pass2_idx), 0)


def _out_index(i):
    t = i - _NB
    j = _NB - 1 - t
    return (jnp.where(i < _NB, _NB - 1, j), 0)


def _const_index(i):
    return (0, 0)


def kernel(X, A, W1, b1, W2, b2):
    b1r = b1.reshape(1, _F)
    b2r = b2.reshape(1, _F)

    y1 = pl.pallas_call(
        _fc1_kernel,
        out_shape=jax.ShapeDtypeStruct((_N, _F), jnp.bfloat16),
    )(X, W1, b1r)

    out = pl.pallas_call(
        _gcn_kernel,
        grid=(2 * _NB,),
        in_specs=[
            pl.BlockSpec((_BM, _N), _a_index),
            pl.BlockSpec((_N, _F), _const_index),
            pl.BlockSpec((_F, _F), _const_index),
            pl.BlockSpec((1, _F), _const_index),
        ],
        out_specs=pl.BlockSpec((_BM, _F), _out_index),
        out_shape=jax.ShapeDtypeStruct((_N, _F), jnp.float32),
        scratch_shapes=[
            pltpu.VMEM((_N, _F), jnp.bfloat16),
            pltpu.VMEM((_S, _BM, _N), jnp.bfloat16),
        ],
        compiler_params=pltpu.CompilerParams(
            dimension_semantics=("arbitrary",),
            vmem_limit_bytes=64 * 1024 * 1024),
    )(A, y1, W2, b2r)
    return out


# speedup vs baseline: 1.0592x; 1.0092x over previous
import jax
import jax.numpy as jnp
from jax.experimental import pallas as pl
from jax.experimental.pallas import tpu as pltpu

_N = 10000
_F = 128
_BM = 200
_NB = _N // _BM
_S = 9


def _fc1_kernel(x_ref, w1_ref, b1_ref, y_ref):
    y1 = jax.lax.dot_general(
        x_ref[...], w1_ref[...],
        (((1,), (1,)), ((), ())), preferred_element_type=jnp.float32)
    y_ref[...] = y1 + b1_ref[...]


def _gcn_kernel(a_ref, y1_ref, w2_ref, b2_ref, out_ref,
                y2f_ref, y2b_ref, cache_ref):
    i = pl.program_id(0)

    @pl.when(i < _NB)
    def _pass1():
        h = jnp.dot(a_ref[...], y1_ref[...],
                    preferred_element_type=jnp.float32)
        h = jnp.maximum(h, 0.0)
        y2 = jax.lax.dot_general(h, w2_ref[...], (((1,), (1,)), ((), ())),
                                 preferred_element_type=jnp.float32)
        y2 = y2 + b2_ref[...]
        y2f_ref[pl.ds(i * _BM, _BM), :] = y2
        y2b_ref[i] = y2.astype(jnp.bfloat16)

        slot = i - (_NB - 1 - _S)

        @pl.when((slot >= 0) & (i <= _NB - 2))
        def _store_cache():
            cache_ref[slot] = a_ref[...].astype(jnp.bfloat16)

    @pl.when(i >= _NB)
    def _pass2():
        t = i - _NB
        from_cache = (t >= 1) & (t <= _S)

        @pl.when(from_cache)
        def _cached():
            out_ref[...] = jnp.dot(cache_ref[_S - t],
                                   y2b_ref[...].reshape(_N, _F),
                                   preferred_element_type=jnp.float32)

        @pl.when(jnp.logical_not(from_cache))
        def _streamed():
            out_ref[...] = jnp.dot(a_ref[...], y2f_ref[...],
                                   preferred_element_type=jnp.float32)


def _a_index(i):
    t = i - _NB
    j = _NB - 1 - t
    pinned = (t >= 0) & (t <= _S)
    pass2_idx = jnp.where(pinned, _NB - 1, j)
    return (jnp.where(i < _NB, i, pass2_idx), 0)


def _out_index(i):
    t = i - _NB
    j = _NB - 1 - t
    return (jnp.where(i < _NB, _NB - 1, j), 0)


def _const_index(i):
    return (0, 0)


def kernel(X, A, W1, b1, W2, b2):
    b1r = b1.reshape(1, _F)
    b2r = b2.reshape(1, _F)

    y1 = pl.pallas_call(
        _fc1_kernel,
        out_shape=jax.ShapeDtypeStruct((_N, _F), jnp.float32),
    )(X, W1, b1r)

    out = pl.pallas_call(
        _gcn_kernel,
        grid=(2 * _NB,),
        in_specs=[
            pl.BlockSpec((_BM, _N), _a_index),
            pl.BlockSpec((_N, _F), _const_index),
            pl.BlockSpec((_F, _F), _const_index),
            pl.BlockSpec((1, _F), _const_index),
        ],
        out_specs=pl.BlockSpec((_BM, _F), _out_index),
        out_shape=jax.ShapeDtypeStruct((_N, _F), jnp.float32),
        scratch_shapes=[
            pltpu.VMEM((_N, _F), jnp.float32),
            pltpu.VMEM((_NB, _BM, _F), jnp.bfloat16),
            pltpu.VMEM((_S, _BM, _N), jnp.bfloat16),
        ],
        compiler_params=pltpu.CompilerParams(
            dimension_semantics=("arbitrary",),
            vmem_limit_bytes=64 * 1024 * 1024),
    )(A, y1, W2, b2r)
    return out


# speedup vs baseline: 1.0700x; 1.0102x over previous
import jax
import jax.numpy as jnp
from jax.experimental import pallas as pl
from jax.experimental.pallas import tpu as pltpu

_N = 10000
_F = 128
_BM = 200
_NB = _N // _BM
_S = 8


def _gcn_kernel(x_ref, a_ref, w1_ref, b1_ref, w2_ref, b2_ref, out_ref,
                y1_ref, y2f_ref, y2b_ref, cache_ref):
    i = pl.program_id(0)

    @pl.when(i == 0)
    def _fc1():
        y1 = jax.lax.dot_general(
            x_ref[...], w1_ref[...],
            (((1,), (1,)), ((), ())), preferred_element_type=jnp.float32)
        y1_ref[...] = y1 + b1_ref[...]

    @pl.when(i < _NB)
    def _pass1():
        h = jnp.dot(a_ref[...], y1_ref[...],
                    preferred_element_type=jnp.float32)
        h = jnp.maximum(h, 0.0)
        y2 = jax.lax.dot_general(h, w2_ref[...], (((1,), (1,)), ((), ())),
                                 preferred_element_type=jnp.float32)
        y2 = y2 + b2_ref[...]
        y2f_ref[pl.ds(i * _BM, _BM), :] = y2
        y2b_ref[i] = y2.astype(jnp.bfloat16)

        slot = i - (_NB - 1 - _S)

        @pl.when((slot >= 0) & (i <= _NB - 2))
        def _store_cache():
            cache_ref[slot] = a_ref[...].astype(jnp.bfloat16)

    @pl.when(i >= _NB)
    def _pass2():
        t = i - _NB
        from_cache = (t >= 1) & (t <= _S)

        @pl.when(from_cache)
        def _cached():
            out_ref[...] = jnp.dot(cache_ref[_S - t],
                                   y2b_ref[...].reshape(_N, _F),
                                   preferred_element_type=jnp.float32)

        @pl.when(jnp.logical_not(from_cache))
        def _streamed():
            out_ref[...] = jnp.dot(a_ref[...], y2f_ref[...],
                                   preferred_element_type=jnp.float32)


def _a_index(i):
    t = i - _NB
    j = _NB - 1 - t
    pinned = (t >= 0) & (t <= _S)
    pass2_idx = jnp.where(pinned, _NB - 1, j)
    return (jnp.where(i < _NB, i, pass2_idx), 0)


def _out_index(i):
    t = i - _NB
    j = _NB - 1 - t
    return (jnp.where(i < _NB, _NB - 1, j), 0)


def _const_index(i):
    return (0, 0)


def kernel(X, A, W1, b1, W2, b2):
    b1r = b1.reshape(1, _F)
    b2r = b2.reshape(1, _F)

    out = pl.pallas_call(
        _gcn_kernel,
        grid=(2 * _NB,),
        in_specs=[
            pl.BlockSpec((_N, _F), _const_index),
            pl.BlockSpec((_BM, _N), _a_index),
            pl.BlockSpec((_F, _F), _const_index),
            pl.BlockSpec((1, _F), _const_index),
            pl.BlockSpec((_F, _F), _const_index),
            pl.BlockSpec((1, _F), _const_index),
        ],
        out_specs=pl.BlockSpec((_BM, _F), _out_index),
        out_shape=jax.ShapeDtypeStruct((_N, _F), jnp.float32),
        scratch_shapes=[
            pltpu.VMEM((_N, _F), jnp.float32),
            pltpu.VMEM((_N, _F), jnp.float32),
            pltpu.VMEM((_NB, _BM, _F), jnp.bfloat16),
            pltpu.VMEM((_S, _BM, _N), jnp.bfloat16),
        ],
        compiler_params=pltpu.CompilerParams(
            dimension_semantics=("arbitrary",),
            vmem_limit_bytes=64 * 1024 * 1024),
    )(X, A, W1, b1r, W2, b2r)
    return out


# speedup vs baseline: 1.1352x; 1.0610x over previous
import jax
import jax.numpy as jnp
from jax.experimental import pallas as pl
from jax.experimental.pallas import tpu as pltpu

_N = 10000
_F = 128
_BM = 200
_NB = _N // _BM
_S = 8
_NS = _NB - _S
_NBUF = 2


def _fetch(a_ref, buf_ref, sem_ref, block, slot):
    return pltpu.make_async_copy(
        a_ref.at[pl.ds(block * _BM, _BM), :], buf_ref.at[slot],
        sem_ref.at[slot])


def _put(outv_ref, out_ref, sem_ref, block, slot):
    return pltpu.make_async_copy(
        outv_ref.at[slot], out_ref.at[pl.ds(block * _BM, _BM), :],
        sem_ref.at[slot])


def _p2_block(t):
    m = t // 5
    cached = (t % 5 == 0) & (t >= 5) & (t <= 5 * _S)
    j = jnp.where(cached, _NS + m - 1, t - jnp.minimum(m, _S))
    return j, cached, m


def _gcn_kernel(x_ref, a_ref, w1_ref, b1_ref, w2_ref, b2_ref, out_ref,
                buf_ref, y1_ref, y2f_ref, y2b_ref, cache_ref, outv_ref,
                in_sem, out_sem):
    for b in range(_NBUF):
        _fetch(a_ref, buf_ref, in_sem, b, b).start()

    y1 = jax.lax.dot_general(
        x_ref[...], w1_ref[...], (((1,), (1,)), ((), ())),
        preferred_element_type=jnp.float32)
    y1_ref[...] = y1 + b1_ref[...]

    def pass1(k, carry):
        sl = jax.lax.rem(k, _NBUF)
        _fetch(a_ref, buf_ref, in_sem, k, sl).wait()
        h = jnp.dot(buf_ref[sl], y1_ref[...],
                    preferred_element_type=jnp.float32)
        h = jnp.maximum(h, 0.0)
        y2 = jax.lax.dot_general(h, w2_ref[...], (((1,), (1,)), ((), ())),
                                 preferred_element_type=jnp.float32)
        y2 = y2 + b2_ref[...]
        y2f_ref[pl.ds(k * _BM, _BM), :] = y2
        y2b_ref[k] = y2.astype(jnp.bfloat16)

        @pl.when(k >= _NB - _S)
        def _store_cache():
            cache_ref[k - (_NB - _S)] = buf_ref[sl].astype(jnp.bfloat16)

        g = k + _NBUF
        nxt = jnp.where(g < _NB, g, g - _NB)
        _fetch(a_ref, buf_ref, in_sem, nxt, sl).start()
        return carry

    jax.lax.fori_loop(0, _NB, pass1, 0)

    def pass2(t, carry):
        j, cached, m = _p2_block(t)
        ov = jax.lax.rem(t, 2)

        @pl.when(t >= 2)
        def _wait_out():
            jprev, _, _ = _p2_block(t - 2)
            _put(outv_ref, out_ref, out_sem, jprev, ov).wait()

        @pl.when(cached)
        def _cached():
            outv_ref[ov] = jnp.dot(cache_ref[m - 1],
                                   y2b_ref[...].reshape(_N, _F),
                                   preferred_element_type=jnp.float32)

        @pl.when(jnp.logical_not(cached))
        def _streamed():
            sl = jax.lax.rem(j, _NBUF)
            _fetch(a_ref, buf_ref, in_sem, j, sl).wait()
            outv_ref[ov] = jnp.dot(buf_ref[sl], y2f_ref[...],
                                   preferred_element_type=jnp.float32)

            @pl.when(j + _NBUF < _NS)
            def _issue():
                _fetch(a_ref, buf_ref, in_sem, j + _NBUF, sl).start()

        _put(outv_ref, out_ref, out_sem, j, ov).start()
        return carry

    jax.lax.fori_loop(0, _NB, pass2, 0)

    for t in (_NB - 2, _NB - 1):
        m = t // 5
        cached = (t % 5 == 0) and (t >= 5) and (t <= 5 * _S)
        j = (_NS + m - 1) if cached else (t - min(m, _S))
        _put(outv_ref, out_ref, out_sem, j, t % 2).wait()


def kernel(X, A, W1, b1, W2, b2):
    b1r = b1.reshape(1, _F)
    b2r = b2.reshape(1, _F)

    out = pl.pallas_call(
        _gcn_kernel,
        in_specs=[
            pl.BlockSpec(memory_space=pltpu.MemorySpace.VMEM),
            pl.BlockSpec(memory_space=pl.ANY),
            pl.BlockSpec(memory_space=pltpu.MemorySpace.VMEM),
            pl.BlockSpec(memory_space=pltpu.MemorySpace.VMEM),
            pl.BlockSpec(memory_space=pltpu.MemorySpace.VMEM),
            pl.BlockSpec(memory_space=pltpu.MemorySpace.VMEM),
        ],
        out_specs=pl.BlockSpec(memory_space=pl.ANY),
        out_shape=jax.ShapeDtypeStruct((_N, _F), jnp.float32),
        scratch_shapes=[
            pltpu.VMEM((_NBUF, _BM, _N), jnp.float32),
            pltpu.VMEM((_N, _F), jnp.float32),
            pltpu.VMEM((_N, _F), jnp.float32),
            pltpu.VMEM((_NB, _BM, _F), jnp.bfloat16),
            pltpu.VMEM((_S, _BM, _N), jnp.bfloat16),
            pltpu.VMEM((2, _BM, _F), jnp.float32),
            pltpu.SemaphoreType.DMA((_NBUF,)),
            pltpu.SemaphoreType.DMA((2,)),
        ],
        compiler_params=pltpu.CompilerParams(
            vmem_limit_bytes=64 * 1024 * 1024),
    )(X, A, W1, b1r, W2, b2r)
    return out


# speedup vs baseline: 1.1445x; 1.0082x over previous
import jax
import jax.numpy as jnp
from jax.experimental import pallas as pl
from jax.experimental.pallas import tpu as pltpu

_N = 10000
_F = 128
_BM = 200
_NB = _N // _BM
_S = 9
_NS = _NB - _S
_NBUF = 2


def _fetch(a_ref, buf_ref, sem_ref, block, slot):
    return pltpu.make_async_copy(
        a_ref.at[pl.ds(block * _BM, _BM), :], buf_ref.at[slot],
        sem_ref.at[slot])


def _put(outv_ref, out_ref, sem_ref, block, slot):
    return pltpu.make_async_copy(
        outv_ref.at[slot], out_ref.at[pl.ds(block * _BM, _BM), :],
        sem_ref.at[slot])


def _p2_block(t):
    m = t // 5
    cached = (t % 5 == 0) & (t >= 5) & (t <= 5 * _S)
    j = jnp.where(cached, _NS + m - 1, t - jnp.minimum(m, _S))
    return j, cached, m


def _gcn_kernel(x_ref, a_ref, w1_ref, b1_ref, w2_ref, b2_ref, out_ref,
                buf_ref, y1_ref, y2b_ref, cache_ref, outv_ref,
                in_sem, out_sem):
    for b in range(_NBUF):
        _fetch(a_ref, buf_ref, in_sem, b, b).start()

    y1 = jax.lax.dot_general(
        x_ref[...], w1_ref[...], (((1,), (1,)), ((), ())),
        preferred_element_type=jnp.float32)
    y1_ref[...] = y1 + b1_ref[...]

    def pass1(k, carry):
        sl = jax.lax.rem(k, _NBUF)
        _fetch(a_ref, buf_ref, in_sem, k, sl).wait()
        h = jnp.dot(buf_ref[sl], y1_ref[...],
                    preferred_element_type=jnp.float32)
        h = jnp.maximum(h, 0.0)
        y2 = jax.lax.dot_general(h, w2_ref[...], (((1,), (1,)), ((), ())),
                                 preferred_element_type=jnp.float32)
        y2 = y2 + b2_ref[...]
        y2b_ref[k] = y2.astype(jnp.bfloat16)

        @pl.when(k >= _NB - _S)
        def _store_cache():
            cache_ref[k - (_NB - _S)] = buf_ref[sl].astype(jnp.bfloat16)

        g = k + _NBUF
        nxt = jnp.where(g < _NB, g, g - _NB)
        _fetch(a_ref, buf_ref, in_sem, nxt, sl).start()
        return carry

    jax.lax.fori_loop(0, _NB, pass1, 0)

    def pass2(t, carry):
        j, cached, m = _p2_block(t)
        ov = jax.lax.rem(t, 2)

        @pl.when(t >= 2)
        def _wait_out():
            jprev, _, _ = _p2_block(t - 2)
            _put(outv_ref, out_ref, out_sem, jprev, ov).wait()

        @pl.when(cached)
        def _cached():
            outv_ref[ov] = jnp.dot(cache_ref[m - 1],
                                   y2b_ref[...].reshape(_N, _F),
                                   preferred_element_type=jnp.float32)

        @pl.when(jnp.logical_not(cached))
        def _streamed():
            sl = jax.lax.rem(j, _NBUF)
            _fetch(a_ref, buf_ref, in_sem, j, sl).wait()
            outv_ref[ov] = jax.lax.dot_general(
                buf_ref[sl], y2b_ref[...].reshape(_N, _F),
                (((1,), (0,)), ((), ())),
                preferred_element_type=jnp.float32)

            @pl.when(j + _NBUF < _NS)
            def _issue():
                _fetch(a_ref, buf_ref, in_sem, j + _NBUF, sl).start()

        _put(outv_ref, out_ref, out_sem, j, ov).start()
        return carry

    jax.lax.fori_loop(0, _NB, pass2, 0)

    for t in (_NB - 2, _NB - 1):
        m = t // 5
        cached = (t % 5 == 0) and (t >= 5) and (t <= 5 * _S)
        j = (_NS + m - 1) if cached else (t - min(m, _S))
        _put(outv_ref, out_ref, out_sem, j, t % 2).wait()


def kernel(X, A, W1, b1, W2, b2):
    b1r = b1.reshape(1, _F)
    b2r = b2.reshape(1, _F)

    out = pl.pallas_call(
        _gcn_kernel,
        in_specs=[
            pl.BlockSpec(memory_space=pltpu.MemorySpace.VMEM),
            pl.BlockSpec(memory_space=pl.ANY),
            pl.BlockSpec(memory_space=pltpu.MemorySpace.VMEM),
            pl.BlockSpec(memory_space=pltpu.MemorySpace.VMEM),
            pl.BlockSpec(memory_space=pltpu.MemorySpace.VMEM),
            pl.BlockSpec(memory_space=pltpu.MemorySpace.VMEM),
        ],
        out_specs=pl.BlockSpec(memory_space=pl.ANY),
        out_shape=jax.ShapeDtypeStruct((_N, _F), jnp.float32),
        scratch_shapes=[
            pltpu.VMEM((_NBUF, _BM, _N), jnp.float32),
            pltpu.VMEM((_N, _F), jnp.float32),
            pltpu.VMEM((_NB, _BM, _F), jnp.bfloat16),
            pltpu.VMEM((_S, _BM, _N), jnp.bfloat16),
            pltpu.VMEM((2, _BM, _F), jnp.float32),
            pltpu.SemaphoreType.DMA((_NBUF,)),
            pltpu.SemaphoreType.DMA((2,)),
        ],
        compiler_params=pltpu.CompilerParams(
            vmem_limit_bytes=64 * 1024 * 1024),
    )(X, A, W1, b1r, W2, b2r)
    return out
